# baseline (speedup 1.0000x reference)
import functools

import jax
import jax.numpy as jnp
from jax import lax
from jax.experimental import pallas as pl
from jax.experimental.pallas import tpu as pltpu
from jax.experimental.pallas import tpu_sc as plsc

L = 2048
N = 32
A = 21
E = 16384
NW = 32
EPW = E // NW
BK = 64
NB = EPW // BK
ROW = A * A + 1
RPW = L // NW
X1W = RPW * A

_mesh = plsc.VectorSubcoreMesh(core_axis_name="c", subcore_axis_name="s")


@functools.partial(
    pl.kernel,
    mesh=_mesh,
    out_type=[
        jax.ShapeDtypeStruct((L * A,), jnp.float32),
        jax.ShapeDtypeStruct((E * 4, 128), jnp.float32),
    ],
    scratch_types=[
        pltpu.VMEM((EPW,), jnp.int32),
        pltpu.VMEM((EPW,), jnp.int32),
        pltpu.VMEM((N,), jnp.float32),
        pltpu.VMEM((N,), jnp.float32),
        pltpu.VMEM((80,), jnp.float32),
        pltpu.VMEM((RPW, N), jnp.int32),
        pltpu.VMEM((BK, N), jnp.int32),
        pltpu.VMEM((BK, N), jnp.int32),
        pltpu.VMEM((BK, N), jnp.int32),
        pltpu.VMEM((BK, N), jnp.int32),
        pltpu.VMEM((BK * 4, 128), jnp.float32),
        pltpu.VMEM((BK * 4, 128), jnp.float32),
        pltpu.VMEM((X1W,), jnp.float32),
        pltpu.SemaphoreType.DMA,
        pltpu.SemaphoreType.DMA,
        pltpu.SemaphoreType.DMA,
        pltpu.SemaphoreType.DMA,
        pltpu.SemaphoreType.DMA,
    ],
    compiler_params=pltpu.CompilerParams(
        needs_layout_passes=False, use_tc_tiling_on_sc=False
    ),
)
def _msa_sc(x_hbm, e0_hbm, e1_hbm, w_hbm, x1_hbm, x2_hbm,
            e0_v, e1_v, w_v, hi, hj, xrow_v,
            ri_a, ri_b, rj_a, rj_b, stage_a, stage_b, x1_st,
            sem_a, sem_b, sem_ra, sem_rb, sem_x1):
    c = lax.axis_index("c")
    s = lax.axis_index("s")
    w = s * 2 + c

    pltpu.sync_copy(e0_hbm.at[pl.ds(w * EPW, EPW)], e0_v)
    pltpu.sync_copy(e1_hbm.at[pl.ds(w * EPW, EPW)], e1_v)
    pltpu.sync_copy(w_hbm, w_v)

    rows = (ri_a, ri_b, rj_a, rj_b)
    rsems = (sem_ra, sem_rb)

    def prefetch(b):
        sl = b % 2
        cpi = pltpu.async_copy(
            x_hbm.at[e0_v.at[pl.ds(b * BK, BK)]], rows[sl], rsems[sl]
        )
        cpj = pltpu.async_copy(
            x_hbm.at[e1_v.at[pl.ds(b * BK, BK)]], rows[2 + sl], rsems[sl]
        )
        return cpi, cpj

    pend_rows = prefetch(0)

    pltpu.sync_copy(x_hbm.at[pl.ds(w * RPW, RPW)], xrow_v)

    iota = lax.iota(jnp.int32, 16)
    zero = jnp.zeros((16,), jnp.float32)

    w0 = w_v[pl.ds(0, 16)]
    w1 = w_v[pl.ds(16, 16)]
    m = jnp.maximum(jnp.max(w0), jnp.max(w1))
    ew0 = jnp.exp(w0 - m)
    ew1 = jnp.exp(w1 - m)
    wsum = jnp.sum(ew0) + jnp.sum(ew1)
    wsm0 = ew0 / wsum
    wsm1 = ew1 / wsum
    wng0 = -wsm0
    wng1 = -wsm1

    def x1_body(r, carry):
        hi[pl.ds(0, 16)] = zero
        hi[pl.ds(16, 16)] = zero
        plsc.addupdate_scatter(hi, [xrow_v[r, pl.ds(0, 16)]], wsm0)
        plsc.addupdate_scatter(hi, [xrow_v[r, pl.ds(16, 16)]], wsm1)
        x1_st[pl.ds(r * A, 16)] = hi[pl.ds(0, 16)]
        x1_st[pl.ds(r * A + 5, 16)] = hi[pl.ds(5, 16)]
        return carry

    lax.fori_loop(0, RPW, x1_body, 0)
    cp_x1 = pltpu.async_copy(
        x1_st, x1_hbm.at[pl.ds(w * X1W, X1W)], sem_x1
    )

    hj[pl.ds(0, 16)] = zero
    hj[pl.ds(16, 16)] = zero
    hj[pl.ds(64, 16)] = zero

    def edge_body_for(stage, ri, rj):
        def edge_body(k, carry):
            xi0 = ri[k, pl.ds(0, 16)]
            xi1 = ri[k, pl.ds(16, 16)]
            xj0 = rj[k, pl.ds(0, 16)]
            xj1 = rj[k, pl.ds(16, 16)]

            hi[pl.ds(0, 16)] = zero
            hi[pl.ds(16, 16)] = zero
            hj[pl.ds(32, 16)] = zero
            hj[pl.ds(48, 16)] = zero
            plsc.addupdate_scatter(hi, [xi0], wsm0)
            plsc.addupdate_scatter(hi, [xi1], wsm1)
            plsc.addupdate_scatter(hj, [xj0 + 32], wng0)
            plsc.addupdate_scatter(hj, [xj1 + 32], wng1)

            hi_v0 = hi[pl.ds(0, 16)]
            hi_v1 = hi[pl.ds(16, 16)]

            sa = []
            for a in range(A):
                sa_s = hi_v0[a] if a < 16 else hi_v1[a - 16]
                sa.append(jnp.full((16,), sa_s, jnp.float32))

            hjx_cache = {}

            def hjx(st):
                if st not in hjx_cache:
                    hjx_cache[st] = hj[pl.ds(st, 16)]
                return hjx_cache[st]

            rowbase = lax.shift_right_logical(k, 3) * 32 + (k & 7)

            for t in range(28):
                a0 = (16 * t) // A
                m0 = 16 * t - A * a0
                lb = A - m0
                rowt = rowbase + (t >> 3) * 8
                col = (t & 7) * 16
                if m0 <= 5:
                    v = sa[a0] * hjx(32 + m0)
                else:
                    v_a = sa[a0] * hjx(32 + m0)
                    if a0 + 1 <= 20:
                        v_b = sa[a0 + 1] * hjx(11 + m0)
                        v = jnp.where(iota < lb, v_a, v_b)
                    else:
                        v = jnp.where(iota < lb, v_a, jnp.float32(0.0))
                stage[rowt, pl.ds(col, 16)] = v

            cc0 = xi0 * A + xj0
            cc1 = xi1 * A + xj1
            rv0 = lax.shift_right_logical(cc0, 7) * 8 + rowbase
            rv1 = lax.shift_right_logical(cc1, 7) * 8 + rowbase
            cv0 = cc0 & 127
            cv1 = cc1 & 127
            old0 = plsc.load_gather(stage, [rv0, cv0])
            old1 = plsc.load_gather(stage, [rv1, cv1])
            plsc.addupdate_scatter(stage, [rv0, cv0], wsm0)
            plsc.addupdate_scatter(stage, [rv1, cv1], wsm1)
            new0 = plsc.load_gather(stage, [rv0, cv0])
            new1 = plsc.load_gather(stage, [rv1, cv1])

            si = jnp.sum(hi_v0 * hi_v0 + hi_v1 * hi_v1)
            hj_a = hjx(32)
            hj_b = hj[pl.ds(48, 16)]
            sj = jnp.sum(hj_a * hj_a + hj_b * hj_b)
            cross = jnp.sum(wsm0 * (old0 + new0) + wsm1 * (old1 + new1))
            ssq = si * sj + cross + jnp.float32(1e-12)

            xv = jnp.full((16,), ssq, jnp.float32)
            bi = lax.bitcast_convert_type(xv, jnp.int32)
            r = lax.bitcast_convert_type(
                jnp.int32(0x5F3759DF) - lax.shift_right_logical(bi, 1),
                jnp.float32,
            )
            half = jnp.float32(0.5) * xv
            for _ in range(3):
                r = r * (jnp.float32(1.5) - half * r * r)
            normv = xv * r
            rown = rowbase + 24
            v26 = stage[rown, pl.ds(48, 16)]
            stage[rown, pl.ds(48, 16)] = jnp.where(iota == 9, normv, v26)
            return carry

        return edge_body

    stages = (stage_a, stage_b)
    osems = (sem_a, sem_b)
    pending = [None, None]
    for b in range(NB):
        sl = b % 2
        if b + 1 < NB:
            nxt = prefetch(b + 1)
        else:
            nxt = None
        for cp in pend_rows:
            cp.wait()
        if pending[sl] is not None:
            pending[sl].wait()
        lax.fori_loop(
            0, BK,
            edge_body_for(stages[sl], rows[sl], rows[2 + sl]),
            0, unroll=2,
        )
        pending[sl] = pltpu.async_copy(
            stages[sl],
            x2_hbm.at[pl.ds((w * NB + b) * BK * 4, BK * 4)],
            osems[sl],
        )
        pend_rows = nxt
    pending[0].wait()
    pending[1].wait()
    cp_x1.wait()


def kernel(x, edge_index, W):
    xf = x[:, :N].astype(jnp.int32)
    e0 = edge_index[0].astype(jnp.int32)
    e1 = edge_index[1].astype(jnp.int32)
    wf = W.astype(jnp.float32).reshape(N)
    x1f, x2f = _msa_sc(xf, e0, e1, wf)
    x2 = (
        x2f.reshape(E // 8, 4, 8, 128)
        .transpose(0, 2, 1, 3)
        .reshape(E, 512)[:, :ROW]
    )
    return x1f.reshape(L, A), x2


# speedup vs baseline: 1.3056x; 1.3056x over previous
import functools

import jax
import jax.numpy as jnp
from jax import lax
from jax.experimental import pallas as pl
from jax.experimental.pallas import tpu as pltpu
from jax.experimental.pallas import tpu_sc as plsc

L = 2048
N = 32
A = 21
E = 16384
NW = 32
EPW = E // NW
BK = 64
NB = EPW // BK
ROW = A * A + 1
RPW = L // NW
X1W = RPW * A

_mesh = plsc.VectorSubcoreMesh(core_axis_name="c", subcore_axis_name="s")


@functools.partial(
    pl.kernel,
    mesh=_mesh,
    out_type=[
        jax.ShapeDtypeStruct((L * A,), jnp.float32),
        jax.ShapeDtypeStruct((E // 8, 4, 8, 128), jnp.float32),
    ],
    scratch_types=[
        pltpu.VMEM((EPW,), jnp.int32),
        pltpu.VMEM((EPW,), jnp.int32),
        pltpu.VMEM((N,), jnp.float32),
        pltpu.VMEM((N,), jnp.float32),
        pltpu.VMEM((80,), jnp.float32),
        pltpu.VMEM((RPW, N), jnp.int32),
        pltpu.VMEM((BK, N), jnp.int32),
        pltpu.VMEM((BK, N), jnp.int32),
        pltpu.VMEM((BK, N), jnp.int32),
        pltpu.VMEM((BK, N), jnp.int32),
        pltpu.VMEM((8, 8, 512), jnp.float32),
        pltpu.VMEM((8, 8, 512), jnp.float32),
        pltpu.VMEM((X1W,), jnp.float32),
        pltpu.SemaphoreType.DMA,
        pltpu.SemaphoreType.DMA,
        pltpu.SemaphoreType.DMA,
        pltpu.SemaphoreType.DMA,
        pltpu.SemaphoreType.DMA,
    ],
    compiler_params=pltpu.CompilerParams(
        needs_layout_passes=False, use_tc_tiling_on_sc=False
    ),
)
def _msa_sc(x_hbm, e0_hbm, e1_hbm, w_hbm, x1_hbm, x2_hbm,
            e0_v, e1_v, w_v, hi, hj, xrow_v,
            ri_a, ri_b, rj_a, rj_b, stage_a, stage_b, x1_st,
            sem_a, sem_b, sem_ra, sem_rb, sem_x1):
    c = lax.axis_index("c")
    s = lax.axis_index("s")
    w = s * 2 + c

    pltpu.sync_copy(e0_hbm.at[pl.ds(w * EPW, EPW)], e0_v)
    pltpu.sync_copy(e1_hbm.at[pl.ds(w * EPW, EPW)], e1_v)
    pltpu.sync_copy(w_hbm, w_v)

    rows = (ri_a, ri_b, rj_a, rj_b)
    rsems = (sem_ra, sem_rb)

    def prefetch(b):
        sl = b % 2
        cpi = pltpu.async_copy(
            x_hbm.at[e0_v.at[pl.ds(b * BK, BK)]], rows[sl], rsems[sl]
        )
        cpj = pltpu.async_copy(
            x_hbm.at[e1_v.at[pl.ds(b * BK, BK)]], rows[2 + sl], rsems[sl]
        )
        return cpi, cpj

    pend_rows = prefetch(0)

    pltpu.sync_copy(x_hbm.at[pl.ds(w * RPW, RPW)], xrow_v)

    iota = lax.iota(jnp.int32, 16)
    zero = jnp.zeros((16,), jnp.float32)

    w0 = w_v[pl.ds(0, 16)]
    w1 = w_v[pl.ds(16, 16)]
    m = jnp.maximum(jnp.max(w0), jnp.max(w1))
    ew0 = jnp.exp(w0 - m)
    ew1 = jnp.exp(w1 - m)
    wsum = jnp.sum(ew0) + jnp.sum(ew1)
    wsm0 = ew0 / wsum
    wsm1 = ew1 / wsum
    wng0 = -wsm0
    wng1 = -wsm1

    def x1_body(r, carry):
        hi[pl.ds(0, 16)] = zero
        hi[pl.ds(16, 16)] = zero
        plsc.addupdate_scatter(hi, [xrow_v[r, pl.ds(0, 16)]], wsm0)
        plsc.addupdate_scatter(hi, [xrow_v[r, pl.ds(16, 16)]], wsm1)
        x1_st[pl.ds(r * A, 16)] = hi[pl.ds(0, 16)]
        x1_st[pl.ds(r * A + 5, 16)] = hi[pl.ds(5, 16)]
        return carry

    lax.fori_loop(0, RPW, x1_body, 0)
    cp_x1 = pltpu.async_copy(
        x1_st, x1_hbm.at[pl.ds(w * X1W, X1W)], sem_x1
    )

    hj[pl.ds(0, 16)] = zero
    hj[pl.ds(16, 16)] = zero
    hj[pl.ds(64, 16)] = zero

    def edge_body_for(stage, ri, rj):
        def edge_body(k, carry):
            xi0 = ri[k, pl.ds(0, 16)]
            xi1 = ri[k, pl.ds(16, 16)]
            xj0 = rj[k, pl.ds(0, 16)]
            xj1 = rj[k, pl.ds(16, 16)]

            hi[pl.ds(0, 16)] = zero
            hi[pl.ds(16, 16)] = zero
            hj[pl.ds(32, 16)] = zero
            hj[pl.ds(48, 16)] = zero
            plsc.addupdate_scatter(hi, [xi0], wsm0)
            plsc.addupdate_scatter(hi, [xi1], wsm1)
            plsc.addupdate_scatter(hj, [xj0 + 32], wng0)
            plsc.addupdate_scatter(hj, [xj1 + 32], wng1)

            hi_v0 = hi[pl.ds(0, 16)]
            hi_v1 = hi[pl.ds(16, 16)]
            hj_lo = hj[pl.ds(32, 16)]
            hj_sh = hj[pl.ds(37, 16)]

            kh = lax.shift_right_logical(k, 3)
            kl = k & 7

            for a in range(A):
                sa_s = hi_v0[a] if a < 16 else hi_v1[a - 16]
                sa = jnp.full((16,), sa_s, jnp.float32)
                stage[kh, kl, pl.ds(a * A, 16)] = sa * hj_lo
                stage[kh, kl, pl.ds(a * A + 5, 16)] = sa * hj_sh

            khv = jnp.full((16,), kh, jnp.int32)
            klv = jnp.full((16,), kl, jnp.int32)
            cc0 = xi0 * A + xj0
            cc1 = xi1 * A + xj1
            old0 = plsc.load_gather(stage, [khv, klv, cc0])
            old1 = plsc.load_gather(stage, [khv, klv, cc1])
            plsc.addupdate_scatter(stage, [khv, klv, cc0], wsm0)
            plsc.addupdate_scatter(stage, [khv, klv, cc1], wsm1)
            new0 = plsc.load_gather(stage, [khv, klv, cc0])
            new1 = plsc.load_gather(stage, [khv, klv, cc1])

            si = jnp.sum(hi_v0 * hi_v0 + hi_v1 * hi_v1)
            hj_b = hj[pl.ds(48, 16)]
            sj = jnp.sum(hj_lo * hj_lo + hj_b * hj_b)
            cross = jnp.sum(wsm0 * (old0 + new0) + wsm1 * (old1 + new1))
            ssq = si * sj + cross + jnp.float32(1e-12)

            xv = jnp.full((16,), ssq, jnp.float32)
            bi = lax.bitcast_convert_type(xv, jnp.int32)
            r = lax.bitcast_convert_type(
                jnp.int32(0x5F3759DF) - lax.shift_right_logical(bi, 1),
                jnp.float32,
            )
            half = jnp.float32(0.5) * xv
            for _ in range(3):
                r = r * (jnp.float32(1.5) - half * r * r)
            normv = xv * r
            v26 = stage[kh, kl, pl.ds(426, 16)]
            stage[kh, kl, pl.ds(426, 16)] = jnp.where(iota == 15, normv, v26)
            return carry

        return edge_body

    stages = (stage_a, stage_b)
    osems = (sem_a, sem_b)
    pending = [None, None]
    for b in range(NB):
        sl = b % 2
        if b + 1 < NB:
            nxt = prefetch(b + 1)
        else:
            nxt = None
        for cp in pend_rows:
            cp.wait()
        if pending[sl] is not None:
            for cp in pending[sl]:
                cp.wait()
        lax.fori_loop(
            0, BK,
            edge_body_for(stages[sl], rows[sl], rows[2 + sl]),
            0, unroll=2,
        )
        kh0 = (w * NB + b) * (BK // 8)
        pending[sl] = [
            pltpu.async_copy(
                stages[sl].at[:, :, pl.ds(tc * 128, 128)],
                x2_hbm.at[pl.ds(kh0, BK // 8), tc],
                osems[sl],
            )
            for tc in range(4)
        ]
        pend_rows = nxt
    for cp in pending[0]:
        cp.wait()
    for cp in pending[1]:
        cp.wait()
    cp_x1.wait()


def kernel(x, edge_index, W):
    xf = x[:, :N].astype(jnp.int32)
    e0 = edge_index[0].astype(jnp.int32)
    e1 = edge_index[1].astype(jnp.int32)
    wf = W.astype(jnp.float32).reshape(N)
    x1f, x2f = _msa_sc(xf, e0, e1, wf)
    x2 = x2f.transpose(0, 2, 1, 3).reshape(E, 512)[:, :ROW]
    return x1f.reshape(L, A), x2


# speedup vs baseline: 1.3060x; 1.0003x over previous
import functools

import jax
import jax.numpy as jnp
from jax import lax
from jax.experimental import pallas as pl
from jax.experimental.pallas import tpu as pltpu
from jax.experimental.pallas import tpu_sc as plsc

L = 2048
N = 32
A = 21
E = 16384
NW = 32
EPW = E // NW
BK = 64
NB = EPW // BK
ROW = A * A + 1
RPW = L // NW
X1W = RPW * A

_mesh = plsc.VectorSubcoreMesh(core_axis_name="c", subcore_axis_name="s")


@functools.partial(
    pl.kernel,
    mesh=_mesh,
    out_type=[
        jax.ShapeDtypeStruct((L, 128), jnp.float32),
        jax.ShapeDtypeStruct((E // 8, 4, 8, 128), jnp.float32),
    ],
    scratch_types=[
        pltpu.VMEM((EPW,), jnp.int32),
        pltpu.VMEM((EPW,), jnp.int32),
        pltpu.VMEM((N,), jnp.float32),
        pltpu.VMEM((N,), jnp.float32),
        pltpu.VMEM((80,), jnp.float32),
        pltpu.VMEM((RPW, N), jnp.int32),
        pltpu.VMEM((BK, N), jnp.int32),
        pltpu.VMEM((BK, N), jnp.int32),
        pltpu.VMEM((BK, N), jnp.int32),
        pltpu.VMEM((BK, N), jnp.int32),
        pltpu.VMEM((8, 8, 512), jnp.float32),
        pltpu.VMEM((8, 8, 512), jnp.float32),
        pltpu.VMEM((RPW, 128), jnp.float32),
        pltpu.SemaphoreType.DMA,
        pltpu.SemaphoreType.DMA,
        pltpu.SemaphoreType.DMA,
        pltpu.SemaphoreType.DMA,
        pltpu.SemaphoreType.DMA,
    ],
    compiler_params=pltpu.CompilerParams(
        needs_layout_passes=False, use_tc_tiling_on_sc=False
    ),
)
def _msa_sc(x_hbm, e0_hbm, e1_hbm, w_hbm, x1_hbm, x2_hbm,
            e0_v, e1_v, w_v, hi, hj, xrow_v,
            ri_a, ri_b, rj_a, rj_b, stage_a, stage_b, x1_st,
            sem_a, sem_b, sem_ra, sem_rb, sem_x1):
    c = lax.axis_index("c")
    s = lax.axis_index("s")
    w = s * 2 + c

    pltpu.sync_copy(e0_hbm.at[pl.ds(w * EPW, EPW)], e0_v)
    pltpu.sync_copy(e1_hbm.at[pl.ds(w * EPW, EPW)], e1_v)
    pltpu.sync_copy(w_hbm, w_v)

    rows = (ri_a, ri_b, rj_a, rj_b)
    rsems = (sem_ra, sem_rb)

    def prefetch(b):
        sl = b % 2
        cpi = pltpu.async_copy(
            x_hbm.at[e0_v.at[pl.ds(b * BK, BK)]], rows[sl], rsems[sl]
        )
        cpj = pltpu.async_copy(
            x_hbm.at[e1_v.at[pl.ds(b * BK, BK)]], rows[2 + sl], rsems[sl]
        )
        return cpi, cpj

    pend_rows = prefetch(0)

    pltpu.sync_copy(x_hbm.at[pl.ds(w * RPW, RPW)], xrow_v)

    iota = lax.iota(jnp.int32, 16)
    zero = jnp.zeros((16,), jnp.float32)

    w0 = w_v[pl.ds(0, 16)]
    w1 = w_v[pl.ds(16, 16)]
    m = jnp.maximum(jnp.max(w0), jnp.max(w1))
    ew0 = jnp.exp(w0 - m)
    ew1 = jnp.exp(w1 - m)
    wsum = jnp.sum(ew0) + jnp.sum(ew1)
    wsm0 = ew0 / wsum
    wsm1 = ew1 / wsum
    wng0 = -wsm0
    wng1 = -wsm1

    def x1_body(r, carry):
        hi[pl.ds(0, 16)] = zero
        hi[pl.ds(16, 16)] = zero
        plsc.addupdate_scatter(hi, [xrow_v[r, pl.ds(0, 16)]], wsm0)
        plsc.addupdate_scatter(hi, [xrow_v[r, pl.ds(16, 16)]], wsm1)
        x1_st[r, pl.ds(0, 16)] = hi[pl.ds(0, 16)]
        x1_st[r, pl.ds(5, 16)] = hi[pl.ds(5, 16)]
        return carry

    lax.fori_loop(0, RPW, x1_body, 0)
    cp_x1 = pltpu.async_copy(
        x1_st, x1_hbm.at[pl.ds(w * RPW, RPW)], sem_x1
    )

    hj[pl.ds(0, 16)] = zero
    hj[pl.ds(16, 16)] = zero
    hj[pl.ds(64, 16)] = zero

    def edge_body_for(stage, ri, rj):
        def edge_body(k, carry):
            xi0 = ri[k, pl.ds(0, 16)]
            xi1 = ri[k, pl.ds(16, 16)]
            xj0 = rj[k, pl.ds(0, 16)]
            xj1 = rj[k, pl.ds(16, 16)]

            hi[pl.ds(0, 16)] = zero
            hi[pl.ds(16, 16)] = zero
            hj[pl.ds(32, 16)] = zero
            hj[pl.ds(48, 16)] = zero
            plsc.addupdate_scatter(hi, [xi0], wsm0)
            plsc.addupdate_scatter(hi, [xi1], wsm1)
            plsc.addupdate_scatter(hj, [xj0 + 32], wng0)
            plsc.addupdate_scatter(hj, [xj1 + 32], wng1)

            hi_v0 = hi[pl.ds(0, 16)]
            hi_v1 = hi[pl.ds(16, 16)]
            hj_lo = hj[pl.ds(32, 16)]
            hj_sh = hj[pl.ds(37, 16)]

            kh = lax.shift_right_logical(k, 3)
            kl = k & 7

            for a in range(A):
                sa_s = hi_v0[a] if a < 16 else hi_v1[a - 16]
                sa = jnp.full((16,), sa_s, jnp.float32)
                stage[kh, kl, pl.ds(a * A, 16)] = sa * hj_lo
                stage[kh, kl, pl.ds(a * A + 5, 16)] = sa * hj_sh

            khv = jnp.full((16,), kh, jnp.int32)
            klv = jnp.full((16,), kl, jnp.int32)
            cc0 = xi0 * A + xj0
            cc1 = xi1 * A + xj1
            old0 = plsc.load_gather(stage, [khv, klv, cc0])
            old1 = plsc.load_gather(stage, [khv, klv, cc1])
            plsc.addupdate_scatter(stage, [khv, klv, cc0], wsm0)
            plsc.addupdate_scatter(stage, [khv, klv, cc1], wsm1)
            new0 = plsc.load_gather(stage, [khv, klv, cc0])
            new1 = plsc.load_gather(stage, [khv, klv, cc1])

            si = jnp.sum(hi_v0 * hi_v0 + hi_v1 * hi_v1)
            hj_b = hj[pl.ds(48, 16)]
            sj = jnp.sum(hj_lo * hj_lo + hj_b * hj_b)
            cross = jnp.sum(wsm0 * (old0 + new0) + wsm1 * (old1 + new1))
            ssq = si * sj + cross + jnp.float32(1e-12)

            xv = jnp.full((16,), ssq, jnp.float32)
            bi = lax.bitcast_convert_type(xv, jnp.int32)
            r = lax.bitcast_convert_type(
                jnp.int32(0x5F3759DF) - lax.shift_right_logical(bi, 1),
                jnp.float32,
            )
            half = jnp.float32(0.5) * xv
            for _ in range(3):
                r = r * (jnp.float32(1.5) - half * r * r)
            normv = xv * r
            v26 = stage[kh, kl, pl.ds(426, 16)]
            stage[kh, kl, pl.ds(426, 16)] = jnp.where(iota == 15, normv, v26)
            return carry

        return edge_body

    stages = (stage_a, stage_b)
    osems = (sem_a, sem_b)
    pending = [None, None]
    for b in range(NB):
        sl = b % 2
        if b + 1 < NB:
            nxt = prefetch(b + 1)
        else:
            nxt = None
        for cp in pend_rows:
            cp.wait()
        if pending[sl] is not None:
            for cp in pending[sl]:
                cp.wait()
        lax.fori_loop(
            0, BK,
            edge_body_for(stages[sl], rows[sl], rows[2 + sl]),
            0, unroll=2,
        )
        kh0 = (w * NB + b) * (BK // 8)
        pending[sl] = [
            pltpu.async_copy(
                stages[sl].at[:, :, pl.ds(tc * 128, 128)],
                x2_hbm.at[pl.ds(kh0, BK // 8), tc],
                osems[sl],
            )
            for tc in range(4)
        ]
        pend_rows = nxt
    for cp in pending[0]:
        cp.wait()
    for cp in pending[1]:
        cp.wait()
    cp_x1.wait()


def kernel(x, edge_index, W):
    xf = x[:, :N].astype(jnp.int32)
    e0 = edge_index[0].astype(jnp.int32)
    e1 = edge_index[1].astype(jnp.int32)
    wf = W.astype(jnp.float32).reshape(N)
    x1f, x2f = _msa_sc(xf, e0, e1, wf)
    x2 = x2f.transpose(0, 2, 1, 3).reshape(E, 512)[:, :ROW]
    return x1f[:, :A], x2
